# baseline (speedup 1.0000x reference)
import functools
import jax
import jax.numpy as jnp
from jax import lax
from jax.experimental import pallas as pl
from jax.experimental.pallas import tpu as pltpu
from jax.experimental.pallas import tpu_sc as plsc

_N = 50000
_E = 800000
_B = 64
_H = 128
_CW = 32
_BK = 400
_EPT = _E // 16
_NB = _EPT // _BK
_NP = 50048
_ZR = 782
_RPT = _NP // 16
_BK2 = 1000
_EPT2 = _E // 32
_NB2 = _E // (32 * _BK2)
_TBLK = 1000
_TGRID = _N // _TBLK

_SC_PARAMS = pltpu.CompilerParams(use_tc_tiling_on_sc=False,
                                  needs_layout_passes=False)


def _sc_mesh():
    return plsc.VectorSubcoreMesh(core_axis_name="c", subcore_axis_name="s")


def _zero_fill(ref, nrow, ncol):
    z = jnp.zeros((16,), jnp.float32)

    def body(r, carry):
        for j in range(ncol // 16):
            ref[r, pl.ds(16 * j, 16)] = z
        return carry

    lax.fori_loop(0, nrow, body, 0)


def _ce_body(eaT_hbm, dst_hbm, out_hbm, dst_v, strip_v, rows_v, acc, gsem):
    cid = lax.axis_index("c")
    sid = lax.axis_index("s")
    _zero_fill(rows_v, _BK2, 16)
    base = sid * _RPT
    for z in range(3):
        pltpu.sync_copy(rows_v, acc.at[pl.ds(base + z * _BK2, _BK2)])
    pltpu.sync_copy(rows_v.at[pl.ds(0, _RPT - 3 * _BK2)],
                    acc.at[pl.ds(base + 3 * _BK2, _RPT - 3 * _BK2)])
    plsc.subcore_barrier()
    wid = cid * 16 + sid
    lane = lax.iota(jnp.int32, 16)

    def batch(i, carry):
        off = (i * 32 + wid) * _BK2
        for d in range(7):
            pltpu.sync_copy(eaT_hbm.at[d, pl.ds(off, _BK2)], strip_v.at[d])
        pltpu.sync_copy(dst_hbm.at[pl.ds(off, _BK2)], dst_v)

        def grp(g, c2):
            rowi = 16 * g + lane
            for d in range(7):
                v = strip_v[d, pl.ds(16 * g, 16)]
                plsc.store_scatter(rows_v, [rowi, jnp.full((16,), d, jnp.int32)], v)
            return c2

        lax.fori_loop(0, _BK2 // 16, grp, 0)
        if _BK2 % 16:
            tail = _BK2 % 16
            rowi = (_BK2 - 16) + lane
            msk = lane >= (16 - tail)
            for d in range(7):
                v = strip_v[d, pl.ds(_BK2 - 16, 16)]
                plsc.store_scatter(rows_v, [rowi, jnp.full((16,), d, jnp.int32)],
                                   v, mask=msk)
        pltpu.sync_copy(rows_v, acc.at[dst_v], add=True)
        return carry

    lax.fori_loop(0, _NB2, batch, 0)
    plsc.subcore_barrier()
    obase = cid * _NP + sid * _RPT
    pltpu.async_copy(acc.at[pl.ds(sid * _RPT, _RPT)],
                     out_hbm.at[pl.ds(obase, _RPT)], gsem).wait()


def _ce_scatter(eaT, dst):
    k = pl.kernel(
        _ce_body,
        out_type=jax.ShapeDtypeStruct((2 * _NP, 16), jnp.float32),
        mesh=_sc_mesh(),
        compiler_params=_SC_PARAMS,
        scratch_types=[
            pltpu.VMEM((_BK2,), jnp.int32),
            pltpu.VMEM((8, _BK2), jnp.float32),
            pltpu.VMEM((_BK2, 16), jnp.float32),
            pltpu.VMEM_SHARED((_NP, 16), jnp.float32),
            pltpu.SemaphoreType.DMA,
        ],
    )
    return k(eaT, dst)


def _scat_body(pr_hbm, src_hbm, dst_hbm, out_hbm,
               src_v, gidx_v, dst_v, rows_v, acc, gsem, ssem):
    cid = lax.axis_index("c")
    sid = lax.axis_index("s")
    ebase = sid * _EPT

    def zfill0():
        z = jnp.zeros((16,), jnp.float32)

        def zbody(r, carry):
            rows_v[0, r, pl.ds(0, 16)] = z
            rows_v[0, r, pl.ds(16, 16)] = z
            return carry

        lax.fori_loop(0, _BK, zbody, 0)

    def load_idx(i, b):
        off = ebase + i * _BK
        pltpu.sync_copy(src_hbm.at[pl.ds(off, _BK)], src_v.at[b])
        pltpu.sync_copy(dst_hbm.at[pl.ds(off, _BK)], dst_v.at[b])

    def comp_gidx(b, c):
        def grp(g, c2):
            sv = src_v[b, pl.ds(16 * g, 16)]
            gidx_v[b, pl.ds(16 * g, 16)] = sv * 4 + c
            return c2

        lax.fori_loop(0, _BK // 16, grp, 0)

    for cl in range(2):
        c = cid * 2 + cl
        zfill0()
        base = sid * _RPT
        zsrc = rows_v.at[0]
        for z in range(7):
            pltpu.sync_copy(zsrc, acc.at[pl.ds(base + z * _BK, _BK)])
        pltpu.sync_copy(zsrc.at[pl.ds(0, _RPT - 7 * _BK)],
                        acc.at[pl.ds(base + 7 * _BK, _RPT - 7 * _BK)])
        plsc.subcore_barrier()

        load_idx(0, 0)
        comp_gidx(0, c)
        pltpu.async_copy(pr_hbm.at[gidx_v.at[0]], rows_v.at[0], gsem)

        def batch(i, carry):
            b = lax.rem(i, 2)
            nb_ = 1 - b

            pltpu.make_async_copy(pr_hbm.at[gidx_v.at[b]], rows_v.at[b],
                                  gsem).wait()
            pltpu.async_copy(rows_v.at[b], acc.at[dst_v.at[b]], ssem, add=True)

            @pl.when(i + 1 < _NB)
            def _():
                @pl.when(i >= 1)
                def _():
                    pltpu.make_async_copy(rows_v.at[nb_],
                                          acc.at[dst_v.at[nb_]], ssem).wait()

                load_idx(i + 1, nb_)
                comp_gidx(nb_, c)
                pltpu.async_copy(pr_hbm.at[gidx_v.at[nb_]], rows_v.at[nb_], gsem)

            return carry

        lax.fori_loop(0, _NB, batch, 0)
        pltpu.make_async_copy(rows_v.at[(_NB - 1) % 2],
                              acc.at[dst_v.at[(_NB - 1) % 2]], ssem).wait()
        plsc.subcore_barrier()
        pltpu.sync_copy(acc.at[pl.ds(sid * _RPT, _RPT)],
                        out_hbm.at[pl.ds(sid * _RPT, _RPT), pl.ds(_CW * c, _CW)])
        plsc.subcore_barrier()


def _scatter128(p, src, dst):
    p_r = p.reshape(4 * _N, _CW)
    k = pl.kernel(
        _scat_body,
        out_type=jax.ShapeDtypeStruct((_NP, _H), jnp.float32),
        mesh=_sc_mesh(),
        compiler_params=_SC_PARAMS,
        scratch_types=[
            pltpu.VMEM((2, _BK), jnp.int32),
            pltpu.VMEM((2, _BK), jnp.int32),
            pltpu.VMEM((2, _BK), jnp.int32),
            pltpu.VMEM((2, _BK, _CW), jnp.float32),
            pltpu.VMEM_SHARED((_NP, _CW), jnp.float32),
            pltpu.SemaphoreType.DMA,
            pltpu.SemaphoreType.DMA,
        ],
    )
    return k(p_r, src, dst)


def _dot(a, b):
    return jax.lax.dot_general(a, b, (((1,), (0,)), ((), ())),
                               preferred_element_type=jnp.float32)


def _stage_a_body(x_ref, wm_ref, ws_ref, xs_ref, p_ref):
    xb = x_ref[...]
    p_ref[...] = _dot(xb, wm_ref[...])
    xs_ref[...] = _dot(xb, ws_ref[...])


def _stage_a(x, Wm1, Ws1):
    blk = _TBLK
    return pl.pallas_call(
        _stage_a_body,
        grid=(_TGRID,),
        in_specs=[
            pl.BlockSpec((blk, 84), lambda i: (i, 0)),
            pl.BlockSpec((84, _H), lambda i: (0, 0)),
            pl.BlockSpec((84, _H), lambda i: (0, 0)),
        ],
        out_specs=[pl.BlockSpec((blk, _H), lambda i: (i, 0)),
                   pl.BlockSpec((blk, _H), lambda i: (i, 0))],
        out_shape=[jax.ShapeDtypeStruct((_N, _H), jnp.float32),
                   jax.ShapeDtypeStruct((_N, _H), jnp.float32)],
    )(x, Wm1, Ws1)


def _ga_body(g_ref, wg_ref, bg_ref, out_ref):
    k = pl.program_id(0)
    part = _dot(g_ref[...], wg_ref[...])

    @pl.when(k == 0)
    def _():
        out_ref[...] = part + bg_ref[...]

    @pl.when(k != 0)
    def _():
        out_ref[...] += part


def _ga_proj(graph_attr, Wg, bg):
    kc = 1152
    return pl.pallas_call(
        _ga_body,
        grid=(10368 // kc,),
        in_specs=[
            pl.BlockSpec((_B, kc), lambda k: (0, k)),
            pl.BlockSpec((kc, _H), lambda k: (k, 0)),
            pl.BlockSpec((1, _H), lambda k: (0, 0)),
        ],
        out_specs=pl.BlockSpec((_B, _H), lambda k: (0, 0)),
        out_shape=jax.ShapeDtypeStruct((_B, _H), jnp.float32),
    )(graph_attr, Wg, bg.reshape(1, _H))


def _mid_body(agg_ref, ce0, ce1, w7_ref, wm_ref, prev_ref, res_ref,
              h_ref, p_ref, *, residual):
    ce = ce0[...] + ce1[...]
    cst = _dot(ce, w7_ref[...])
    h = jax.nn.relu(agg_ref[...] + cst + prev_ref[...])
    if residual:
        h = h + res_ref[...]
    h_ref[...] = h
    p_ref[...] = _dot(h, wm_ref[...])


def _stage_mid(agg, ce2, W7, Wm_next, prev, res, residual):
    blk = _TBLK
    body = functools.partial(_mid_body, residual=residual)
    return pl.pallas_call(
        body,
        grid=(_TGRID,),
        in_specs=[pl.BlockSpec((blk, _H), lambda i: (i, 0))] +
                 [pl.BlockSpec((blk, 16), lambda i: (i, 0))] * 2 +
                 [pl.BlockSpec((16, _H), lambda i: (0, 0)),
                  pl.BlockSpec((_H, _H), lambda i: (0, 0)),
                  pl.BlockSpec((blk, _H), lambda i: (i, 0)),
                  pl.BlockSpec((blk, _H), lambda i: (i, 0))],
        out_specs=[pl.BlockSpec((blk, _H), lambda i: (i, 0)),
                   pl.BlockSpec((blk, _H), lambda i: (i, 0))],
        out_shape=[jax.ShapeDtypeStruct((_N, _H), jnp.float32),
                   jax.ShapeDtypeStruct((_N, _H), jnp.float32)],
    )(agg, ce2[0], ce2[1], W7, Wm_next, prev, res)


def _pool_body(agg_ref, ce0, ce1, w7_ref, x2_ref, b_ref, psum_ref, cnt_ref):
    i = pl.program_id(0)
    ce = ce0[...] + ce1[...]
    cst = _dot(ce, w7_ref[...])
    x2 = x2_ref[...]
    x3 = jax.nn.relu(agg_ref[...] + cst + x2) + x2
    b = b_ref[0, 0, :]
    iot = lax.broadcasted_iota(jnp.int32, (_TBLK, _B), 1)
    oh = (b[:, None] == iot).astype(jnp.float32)
    psum = jax.lax.dot_general(oh, x3, (((0,), (0,)), ((), ())),
                               preferred_element_type=jnp.float32)
    cnt = jax.lax.dot_general(oh, jnp.ones((_TBLK, _H), jnp.float32),
                              (((0,), (0,)), ((), ())),
                              preferred_element_type=jnp.float32)

    @pl.when(i == 0)
    def _():
        psum_ref[...] = psum
        cnt_ref[...] = cnt

    @pl.when(i != 0)
    def _():
        psum_ref[...] += psum
        cnt_ref[...] += cnt


def _stage_pool(agg, ce2, W7, x2, batch3):
    blk = _TBLK
    return pl.pallas_call(
        _pool_body,
        grid=(_TGRID,),
        in_specs=[pl.BlockSpec((blk, _H), lambda i: (i, 0))] +
                 [pl.BlockSpec((blk, 16), lambda i: (i, 0))] * 2 +
                 [pl.BlockSpec((16, _H), lambda i: (0, 0)),
                  pl.BlockSpec((blk, _H), lambda i: (i, 0)),
                  pl.BlockSpec((1, 1, blk), lambda i: (i, 0, 0))],
        out_specs=[pl.BlockSpec((_B, _H), lambda i: (0, 0)),
                   pl.BlockSpec((_B, _H), lambda i: (0, 0))],
        out_shape=[jax.ShapeDtypeStruct((_B, _H), jnp.float32),
                   jax.ShapeDtypeStruct((_B, _H), jnp.float32)],
    )(agg, ce2[0], ce2[1], W7, x2, batch3)


def _head_body(psum_ref, cnt_ref, ga_ref, wcp_ref, wcg_ref, bc_ref,
               wl_ref, bl_ref, out_ref):
    cnt = jnp.maximum(cnt_ref[...], 1.0)
    pooled = psum_ref[...] / cnt
    h = jax.nn.relu(_dot(pooled, wcp_ref[...]) + _dot(ga_ref[...], wcg_ref[...])
                    + bc_ref[...])
    out_ref[...] = _dot(h, wl_ref[...]) + bl_ref[...]


def _head(psum, cnt, ga, Wc, bc, Wl, bl):
    return pl.pallas_call(
        _head_body,
        out_shape=jax.ShapeDtypeStruct((_B, 1), jnp.float32),
    )(psum, cnt, ga, Wc[:_H], Wc[_H:], bc.reshape(1, _H), Wl,
      bl.reshape(1, 1))


def kernel(x, edge_index, edge_attr, batch, graph_attr,
           Wm1, bm1, We1, be1, Ws1,
           Wm2, bm2, We2, be2,
           Wm3, bm3, We3, be3,
           Wg, bg, Wc, bc, Wl, bl):
    src = edge_index[0]
    dst = edge_index[1]

    eaT = jnp.concatenate(
        [edge_attr.T, jnp.ones((1, _E), jnp.float32),
         jnp.zeros((1, _E), jnp.float32)], axis=0)

    def w7(We, bm, be):
        return jnp.concatenate(
            [We, (bm + be).reshape(1, _H), jnp.zeros((16 - 7, _H), jnp.float32)],
            axis=0)

    W7_1, W7_2, W7_3 = w7(We1, bm1, be1), w7(We2, bm2, be2), w7(We3, bm3, be3)

    ce_part = _ce_scatter(eaT, dst)
    ce2 = (ce_part[:_N], ce_part[_NP:_NP + _N])

    xs, p1 = _stage_a(x, Wm1, Ws1)
    ga = _ga_proj(graph_attr, Wg, bg)

    agg1 = _scatter128(p1, src, dst)
    x1, p2 = _stage_mid(agg1[:_N], ce2, W7_1, Wm2, xs, xs, residual=False)

    agg2 = _scatter128(p2, src, dst)
    x2, p3 = _stage_mid(agg2[:_N], ce2, W7_2, Wm3, x1, x1, residual=True)

    agg3 = _scatter128(p3, src, dst)
    batch3 = batch.reshape(_TGRID, 1, _TBLK)
    psum, cnt = _stage_pool(agg3[:_N], ce2, W7_3, x2, batch3)

    return _head(psum, cnt, ga, Wc, bc, Wl, bl)


# speedup vs baseline: 1.1853x; 1.1853x over previous
import functools
import jax
import jax.numpy as jnp
from jax import lax
from jax.experimental import pallas as pl
from jax.experimental.pallas import tpu as pltpu
from jax.experimental.pallas import tpu_sc as plsc

_N = 50000
_E = 800000
_B = 64
_H = 128
_CW = 32
_BK = 400
_EPT = _E // 16
_NB = _EPT // _BK
_NP = 50048
_ZR = 782
_RPT = _NP // 16
_BK2 = 1000
_EPT2 = _E // 32
_NB2 = _E // (32 * _BK2)
_TBLK = 1000
_TGRID = _N // _TBLK

_SC_PARAMS = pltpu.CompilerParams(use_tc_tiling_on_sc=False,
                                  needs_layout_passes=False)


def _sc_mesh():
    return plsc.VectorSubcoreMesh(core_axis_name="c", subcore_axis_name="s")


def _zero_fill(ref, nrow, ncol):
    z = jnp.zeros((16,), jnp.float32)

    def body(r, carry):
        for j in range(ncol // 16):
            ref[r, pl.ds(16 * j, 16)] = z
        return carry

    lax.fori_loop(0, nrow, body, 0)


def _ce_body(eaT_hbm, dst_hbm, out_hbm, dst_v, strip_v, rows_v, acc, gsem):
    cid = lax.axis_index("c")
    sid = lax.axis_index("s")
    _zero_fill(rows_v, _BK2, 16)
    base = sid * _RPT
    for z in range(3):
        pltpu.sync_copy(rows_v, acc.at[pl.ds(base + z * _BK2, _BK2)])
    pltpu.sync_copy(rows_v.at[pl.ds(0, _RPT - 3 * _BK2)],
                    acc.at[pl.ds(base + 3 * _BK2, _RPT - 3 * _BK2)])
    plsc.subcore_barrier()
    wid = cid * 16 + sid
    lane = lax.iota(jnp.int32, 16)

    def batch(i, carry):
        off = (i * 32 + wid) * _BK2
        for d in range(7):
            pltpu.sync_copy(eaT_hbm.at[d, pl.ds(off, _BK2)], strip_v.at[d])
        pltpu.sync_copy(dst_hbm.at[pl.ds(off, _BK2)], dst_v)

        def grp(g, c2):
            rowi = 16 * g + lane
            for d in range(7):
                v = strip_v[d, pl.ds(16 * g, 16)]
                plsc.store_scatter(rows_v, [rowi, jnp.full((16,), d, jnp.int32)], v)
            return c2

        lax.fori_loop(0, _BK2 // 16, grp, 0)
        if _BK2 % 16:
            tail = _BK2 % 16
            rowi = (_BK2 - 16) + lane
            msk = lane >= (16 - tail)
            for d in range(7):
                v = strip_v[d, pl.ds(_BK2 - 16, 16)]
                plsc.store_scatter(rows_v, [rowi, jnp.full((16,), d, jnp.int32)],
                                   v, mask=msk)
        pltpu.sync_copy(rows_v, acc.at[dst_v], add=True)
        return carry

    lax.fori_loop(0, _NB2, batch, 0)
    plsc.subcore_barrier()
    obase = cid * _NP + sid * _RPT
    pltpu.async_copy(acc.at[pl.ds(sid * _RPT, _RPT)],
                     out_hbm.at[pl.ds(obase, _RPT)], gsem).wait()


def _ce_scatter(eaT, dst):
    k = pl.kernel(
        _ce_body,
        out_type=jax.ShapeDtypeStruct((2 * _NP, 16), jnp.float32),
        mesh=_sc_mesh(),
        compiler_params=_SC_PARAMS,
        scratch_types=[
            pltpu.VMEM((_BK2,), jnp.int32),
            pltpu.VMEM((8, _BK2), jnp.float32),
            pltpu.VMEM((_BK2, 16), jnp.float32),
            pltpu.VMEM_SHARED((_NP, 16), jnp.float32),
            pltpu.SemaphoreType.DMA,
        ],
    )
    return k(eaT, dst)


def _scat_body(pr_hbm, src_hbm, dst_hbm, out_hbm,
               src_v, gidx_v, dst_v, rows_v, acc, gsem, ssem):
    cid = lax.axis_index("c")
    sid = lax.axis_index("s")
    ebase = sid * _EPT

    def zfill0():
        z = jnp.zeros((16,), jnp.float32)

        def zbody(r, carry):
            rows_v[0, r, pl.ds(0, 16)] = z
            rows_v[0, r, pl.ds(16, 16)] = z
            return carry

        lax.fori_loop(0, _BK, zbody, 0)

    def load_idx(i, b):
        off = ebase + i * _BK
        pltpu.sync_copy(src_hbm.at[pl.ds(off, _BK)], src_v.at[b])
        pltpu.sync_copy(dst_hbm.at[pl.ds(off, _BK)], dst_v.at[b])

    def comp_gidx(b, c):
        def grp(g, c2):
            sv = src_v[b, pl.ds(16 * g, 16)]
            gidx_v[b, pl.ds(16 * g, 16)] = sv * 4 + c
            return c2

        lax.fori_loop(0, _BK // 16, grp, 0)

    for cl in range(2):
        c = cid * 2 + cl
        zfill0()
        base = sid * _RPT
        zsrc = rows_v.at[0]
        for z in range(7):
            pltpu.sync_copy(zsrc, acc.at[pl.ds(base + z * _BK, _BK)])
        pltpu.sync_copy(zsrc.at[pl.ds(0, _RPT - 7 * _BK)],
                        acc.at[pl.ds(base + 7 * _BK, _RPT - 7 * _BK)])
        plsc.subcore_barrier()

        load_idx(0, 0)
        comp_gidx(0, c)
        pltpu.async_copy(pr_hbm.at[gidx_v.at[0]], rows_v.at[0], gsem)

        def batch(i, carry):
            b = lax.rem(i, 2)
            nb_ = 1 - b

            pltpu.make_async_copy(pr_hbm.at[gidx_v.at[b]], rows_v.at[b],
                                  gsem).wait()
            pltpu.async_copy(rows_v.at[b], acc.at[dst_v.at[b]], ssem.at[b], add=True)

            @pl.when(i + 1 < _NB)
            def _():
                @pl.when(i >= 1)
                def _():
                    pltpu.make_async_copy(rows_v.at[nb_],
                                          acc.at[dst_v.at[nb_]], ssem.at[nb_]).wait()

                load_idx(i + 1, nb_)
                comp_gidx(nb_, c)
                pltpu.async_copy(pr_hbm.at[gidx_v.at[nb_]], rows_v.at[nb_], gsem)

            return carry

        lax.fori_loop(0, _NB, batch, 0)
        pltpu.make_async_copy(rows_v.at[(_NB - 1) % 2],
                              acc.at[dst_v.at[(_NB - 1) % 2]],
                              ssem.at[(_NB - 1) % 2]).wait()
        plsc.subcore_barrier()
        pltpu.sync_copy(acc.at[pl.ds(sid * _RPT, _RPT)],
                        out_hbm.at[pl.ds(sid * _RPT, _RPT), pl.ds(_CW * c, _CW)])
        plsc.subcore_barrier()


def _scatter128(p, src, dst):
    p_r = p.reshape(4 * _N, _CW)
    k = pl.kernel(
        _scat_body,
        out_type=jax.ShapeDtypeStruct((_NP, _H), jnp.float32),
        mesh=_sc_mesh(),
        compiler_params=_SC_PARAMS,
        scratch_types=[
            pltpu.VMEM((2, _BK), jnp.int32),
            pltpu.VMEM((2, _BK), jnp.int32),
            pltpu.VMEM((2, _BK), jnp.int32),
            pltpu.VMEM((2, _BK, _CW), jnp.float32),
            pltpu.VMEM_SHARED((_NP, _CW), jnp.float32),
            pltpu.SemaphoreType.DMA,
            pltpu.SemaphoreType.DMA((2,)),
        ],
    )
    return k(p_r, src, dst)


def _dot(a, b):
    return jax.lax.dot_general(a, b, (((1,), (0,)), ((), ())),
                               preferred_element_type=jnp.float32)


def _stage_a_body(x_ref, wm_ref, ws_ref, xs_ref, p_ref):
    xb = x_ref[...]
    p_ref[...] = _dot(xb, wm_ref[...])
    xs_ref[...] = _dot(xb, ws_ref[...])


def _stage_a(x, Wm1, Ws1):
    blk = _TBLK
    return pl.pallas_call(
        _stage_a_body,
        grid=(_TGRID,),
        in_specs=[
            pl.BlockSpec((blk, 84), lambda i: (i, 0)),
            pl.BlockSpec((84, _H), lambda i: (0, 0)),
            pl.BlockSpec((84, _H), lambda i: (0, 0)),
        ],
        out_specs=[pl.BlockSpec((blk, _H), lambda i: (i, 0)),
                   pl.BlockSpec((blk, _H), lambda i: (i, 0))],
        out_shape=[jax.ShapeDtypeStruct((_N, _H), jnp.float32),
                   jax.ShapeDtypeStruct((_N, _H), jnp.float32)],
    )(x, Wm1, Ws1)


def _ga_body(g_ref, wg_ref, bg_ref, out_ref):
    k = pl.program_id(0)
    part = _dot(g_ref[...], wg_ref[...])

    @pl.when(k == 0)
    def _():
        out_ref[...] = part + bg_ref[...]

    @pl.when(k != 0)
    def _():
        out_ref[...] += part


def _ga_proj(graph_attr, Wg, bg):
    kc = 1152
    return pl.pallas_call(
        _ga_body,
        grid=(10368 // kc,),
        in_specs=[
            pl.BlockSpec((_B, kc), lambda k: (0, k)),
            pl.BlockSpec((kc, _H), lambda k: (k, 0)),
            pl.BlockSpec((1, _H), lambda k: (0, 0)),
        ],
        out_specs=pl.BlockSpec((_B, _H), lambda k: (0, 0)),
        out_shape=jax.ShapeDtypeStruct((_B, _H), jnp.float32),
    )(graph_attr, Wg, bg.reshape(1, _H))


def _mid_body(agg_ref, ce0, ce1, w7_ref, wm_ref, prev_ref, res_ref,
              h_ref, p_ref, *, residual):
    ce = ce0[...] + ce1[...]
    cst = _dot(ce, w7_ref[...])
    h = jax.nn.relu(agg_ref[...] + cst + prev_ref[...])
    if residual:
        h = h + res_ref[...]
    h_ref[...] = h
    p_ref[...] = _dot(h, wm_ref[...])


def _stage_mid(agg, ce2, W7, Wm_next, prev, res, residual):
    blk = _TBLK
    body = functools.partial(_mid_body, residual=residual)
    return pl.pallas_call(
        body,
        grid=(_TGRID,),
        in_specs=[pl.BlockSpec((blk, _H), lambda i: (i, 0))] +
                 [pl.BlockSpec((blk, 16), lambda i: (i, 0))] * 2 +
                 [pl.BlockSpec((16, _H), lambda i: (0, 0)),
                  pl.BlockSpec((_H, _H), lambda i: (0, 0)),
                  pl.BlockSpec((blk, _H), lambda i: (i, 0)),
                  pl.BlockSpec((blk, _H), lambda i: (i, 0))],
        out_specs=[pl.BlockSpec((blk, _H), lambda i: (i, 0)),
                   pl.BlockSpec((blk, _H), lambda i: (i, 0))],
        out_shape=[jax.ShapeDtypeStruct((_N, _H), jnp.float32),
                   jax.ShapeDtypeStruct((_N, _H), jnp.float32)],
    )(agg, ce2[0], ce2[1], W7, Wm_next, prev, res)


def _pool_body(agg_ref, ce0, ce1, w7_ref, x2_ref, b_ref, psum_ref, cnt_ref):
    i = pl.program_id(0)
    ce = ce0[...] + ce1[...]
    cst = _dot(ce, w7_ref[...])
    x2 = x2_ref[...]
    x3 = jax.nn.relu(agg_ref[...] + cst + x2) + x2
    b = b_ref[0, 0, :]
    iot = lax.broadcasted_iota(jnp.int32, (_TBLK, _B), 1)
    oh = (b[:, None] == iot).astype(jnp.float32)
    psum = jax.lax.dot_general(oh, x3, (((0,), (0,)), ((), ())),
                               preferred_element_type=jnp.float32)
    cnt = jax.lax.dot_general(oh, jnp.ones((_TBLK, _H), jnp.float32),
                              (((0,), (0,)), ((), ())),
                              preferred_element_type=jnp.float32)

    @pl.when(i == 0)
    def _():
        psum_ref[...] = psum
        cnt_ref[...] = cnt

    @pl.when(i != 0)
    def _():
        psum_ref[...] += psum
        cnt_ref[...] += cnt


def _stage_pool(agg, ce2, W7, x2, batch3):
    blk = _TBLK
    return pl.pallas_call(
        _pool_body,
        grid=(_TGRID,),
        in_specs=[pl.BlockSpec((blk, _H), lambda i: (i, 0))] +
                 [pl.BlockSpec((blk, 16), lambda i: (i, 0))] * 2 +
                 [pl.BlockSpec((16, _H), lambda i: (0, 0)),
                  pl.BlockSpec((blk, _H), lambda i: (i, 0)),
                  pl.BlockSpec((1, 1, blk), lambda i: (i, 0, 0))],
        out_specs=[pl.BlockSpec((_B, _H), lambda i: (0, 0)),
                   pl.BlockSpec((_B, _H), lambda i: (0, 0))],
        out_shape=[jax.ShapeDtypeStruct((_B, _H), jnp.float32),
                   jax.ShapeDtypeStruct((_B, _H), jnp.float32)],
    )(agg, ce2[0], ce2[1], W7, x2, batch3)


def _head_body(psum_ref, cnt_ref, ga_ref, wcp_ref, wcg_ref, bc_ref,
               wl_ref, bl_ref, out_ref):
    cnt = jnp.maximum(cnt_ref[...], 1.0)
    pooled = psum_ref[...] / cnt
    h = jax.nn.relu(_dot(pooled, wcp_ref[...]) + _dot(ga_ref[...], wcg_ref[...])
                    + bc_ref[...])
    out_ref[...] = _dot(h, wl_ref[...]) + bl_ref[...]


def _head(psum, cnt, ga, Wc, bc, Wl, bl):
    return pl.pallas_call(
        _head_body,
        out_shape=jax.ShapeDtypeStruct((_B, 1), jnp.float32),
    )(psum, cnt, ga, Wc[:_H], Wc[_H:], bc.reshape(1, _H), Wl,
      bl.reshape(1, 1))


def kernel(x, edge_index, edge_attr, batch, graph_attr,
           Wm1, bm1, We1, be1, Ws1,
           Wm2, bm2, We2, be2,
           Wm3, bm3, We3, be3,
           Wg, bg, Wc, bc, Wl, bl):
    src = edge_index[0]
    dst = edge_index[1]

    eaT = jnp.concatenate(
        [edge_attr.T, jnp.ones((1, _E), jnp.float32),
         jnp.zeros((1, _E), jnp.float32)], axis=0)

    def w7(We, bm, be):
        return jnp.concatenate(
            [We, (bm + be).reshape(1, _H), jnp.zeros((16 - 7, _H), jnp.float32)],
            axis=0)

    W7_1, W7_2, W7_3 = w7(We1, bm1, be1), w7(We2, bm2, be2), w7(We3, bm3, be3)

    ce_part = _ce_scatter(eaT, dst)
    ce2 = (ce_part[:_N], ce_part[_NP:_NP + _N])

    xs, p1 = _stage_a(x, Wm1, Ws1)
    ga = _ga_proj(graph_attr, Wg, bg)

    agg1 = _scatter128(p1, src, dst)
    x1, p2 = _stage_mid(agg1[:_N], ce2, W7_1, Wm2, xs, xs, residual=False)

    agg2 = _scatter128(p2, src, dst)
    x2, p3 = _stage_mid(agg2[:_N], ce2, W7_2, Wm3, x1, x1, residual=True)

    agg3 = _scatter128(p3, src, dst)
    batch3 = batch.reshape(_TGRID, 1, _TBLK)
    psum, cnt = _stage_pool(agg3[:_N], ce2, W7_3, x2, batch3)

    return _head(psum, cnt, ga, Wc, bc, Wl, bl)


# speedup vs baseline: 1.4783x; 1.2472x over previous
import functools
import jax
import jax.numpy as jnp
from jax import lax
from jax.experimental import pallas as pl
from jax.experimental.pallas import tpu as pltpu
from jax.experimental.pallas import tpu_sc as plsc

_N = 50000
_E = 800000
_B = 64
_H = 128
_CW = 32
_BK = 400
_EPT = _E // 16
_NB = _EPT // _BK
_NP = 50048
_ZR = 782
_RPT = _NP // 16
_BK2 = 1000
_EPT2 = _E // 32
_NB2 = _E // (32 * _BK2)
_TBLK = 1000
_TGRID = _N // _TBLK

_SC_PARAMS = pltpu.CompilerParams(use_tc_tiling_on_sc=False,
                                  needs_layout_passes=False)


def _sc_mesh():
    return plsc.VectorSubcoreMesh(core_axis_name="c", subcore_axis_name="s")


def _zero_fill(ref, nrow, ncol):
    z = jnp.zeros((16,), jnp.float32)

    def body(r, carry):
        for j in range(ncol // 16):
            ref[r, pl.ds(16 * j, 16)] = z
        return carry

    lax.fori_loop(0, nrow, body, 0)


def _ce_body(eaT_hbm, dst_hbm, out_hbm, dst_v, strip_v, rows_v, acc, gsem):
    cid = lax.axis_index("c")
    sid = lax.axis_index("s")
    _zero_fill(rows_v, _BK2, 16)
    base = sid * _RPT
    for z in range(3):
        pltpu.sync_copy(rows_v, acc.at[pl.ds(base + z * _BK2, _BK2)])
    pltpu.sync_copy(rows_v.at[pl.ds(0, _RPT - 3 * _BK2)],
                    acc.at[pl.ds(base + 3 * _BK2, _RPT - 3 * _BK2)])
    plsc.subcore_barrier()
    wid = cid * 16 + sid
    lane = lax.iota(jnp.int32, 16)

    def batch(i, carry):
        off = (i * 32 + wid) * _BK2
        for d in range(7):
            pltpu.sync_copy(eaT_hbm.at[d, pl.ds(off, _BK2)], strip_v.at[d])
        pltpu.sync_copy(dst_hbm.at[pl.ds(off, _BK2)], dst_v)

        def grp(g, c2):
            rowi = 16 * g + lane
            for d in range(7):
                v = strip_v[d, pl.ds(16 * g, 16)]
                plsc.store_scatter(rows_v, [rowi, jnp.full((16,), d, jnp.int32)], v)
            return c2

        lax.fori_loop(0, _BK2 // 16, grp, 0)
        if _BK2 % 16:
            tail = _BK2 % 16
            rowi = (_BK2 - 16) + lane
            msk = lane >= (16 - tail)
            for d in range(7):
                v = strip_v[d, pl.ds(_BK2 - 16, 16)]
                plsc.store_scatter(rows_v, [rowi, jnp.full((16,), d, jnp.int32)],
                                   v, mask=msk)
        pltpu.sync_copy(rows_v, acc.at[dst_v], add=True)
        return carry

    lax.fori_loop(0, _NB2, batch, 0)
    plsc.subcore_barrier()
    obase = cid * _NP + sid * _RPT
    pltpu.async_copy(acc.at[pl.ds(sid * _RPT, _RPT)],
                     out_hbm.at[pl.ds(obase, _RPT)], gsem).wait()


def _ce_scatter(eaT, dst):
    k = pl.kernel(
        _ce_body,
        out_type=jax.ShapeDtypeStruct((2 * _NP, 16), jnp.float32),
        mesh=_sc_mesh(),
        compiler_params=_SC_PARAMS,
        scratch_types=[
            pltpu.VMEM((_BK2,), jnp.int32),
            pltpu.VMEM((8, _BK2), jnp.float32),
            pltpu.VMEM((_BK2, 16), jnp.float32),
            pltpu.VMEM_SHARED((_NP, 16), jnp.float32),
            pltpu.SemaphoreType.DMA,
        ],
    )
    return k(eaT, dst)


def _scat_body(pr_hbm, g4_hbm, dst_hbm, out_hbm,
               gidx_v, dst_v, rows_v, acc, gsem, isem):
    cid = lax.axis_index("c")
    sid = lax.axis_index("s")
    ebase = sid * _EPT

    def zfill0():
        z = jnp.zeros((16,), jnp.float32)

        def zbody(r, carry):
            rows_v[0, r, pl.ds(0, 16)] = z
            rows_v[0, r, pl.ds(16, 16)] = z
            return carry

        lax.fori_loop(0, _BK, zbody, 0)

    for cl in range(2):
        c = cid * 2 + cl
        zfill0()
        base = sid * _RPT
        zsrc = rows_v.at[0]
        for z in range(7):
            pltpu.sync_copy(zsrc, acc.at[pl.ds(base + z * _BK, _BK)])
        pltpu.sync_copy(zsrc.at[pl.ds(0, _RPT - 7 * _BK)],
                        acc.at[pl.ds(base + 7 * _BK, _RPT - 7 * _BK)])
        plsc.subcore_barrier()

        pltpu.sync_copy(g4_hbm.at[c, pl.ds(ebase, _BK)], gidx_v.at[0])
        pltpu.sync_copy(dst_hbm.at[pl.ds(ebase, _BK)], dst_v.at[0])
        pltpu.async_copy(pr_hbm.at[gidx_v.at[0]], rows_v.at[0], gsem)

        def batch(i, carry):
            b = lax.rem(i, 2)
            nb_ = 1 - b
            off = ebase + (i + 1) * _BK

            @pl.when(i + 1 < _NB)
            def _():
                pltpu.async_copy(g4_hbm.at[c, pl.ds(off, _BK)],
                                 gidx_v.at[nb_], isem)
                pltpu.async_copy(dst_hbm.at[pl.ds(off, _BK)], dst_v.at[nb_],
                                 isem)

            pltpu.make_async_copy(pr_hbm.at[gidx_v.at[b]], rows_v.at[b],
                                  gsem).wait()

            @pl.when(i + 1 < _NB)
            def _():
                pltpu.make_async_copy(g4_hbm.at[c, pl.ds(off, _BK)],
                                      gidx_v.at[nb_], isem).wait()
                pltpu.make_async_copy(dst_hbm.at[pl.ds(off, _BK)],
                                      dst_v.at[nb_], isem).wait()
                pltpu.async_copy(pr_hbm.at[gidx_v.at[nb_]], rows_v.at[nb_], gsem)

            pltpu.sync_copy(rows_v.at[b], acc.at[dst_v.at[b]], add=True)
            return carry

        lax.fori_loop(0, _NB, batch, 0)
        plsc.subcore_barrier()
        pltpu.sync_copy(acc.at[pl.ds(sid * _RPT, _RPT)],
                        out_hbm.at[pl.ds(sid * _RPT, _RPT), pl.ds(_CW * c, _CW)])
        plsc.subcore_barrier()


def _scatter128(p, g4, dst):
    p_r = p.reshape(4 * _N, _CW)
    k = pl.kernel(
        _scat_body,
        out_type=jax.ShapeDtypeStruct((_NP, _H), jnp.float32),
        mesh=_sc_mesh(),
        compiler_params=_SC_PARAMS,
        scratch_types=[
            pltpu.VMEM((2, _BK), jnp.int32),
            pltpu.VMEM((2, _BK), jnp.int32),
            pltpu.VMEM((2, _BK, _CW), jnp.float32),
            pltpu.VMEM_SHARED((_NP, _CW), jnp.float32),
            pltpu.SemaphoreType.DMA,
            pltpu.SemaphoreType.DMA,
        ],
    )
    return k(p_r, g4, dst)


def _dot(a, b):
    return jax.lax.dot_general(a, b, (((1,), (0,)), ((), ())),
                               preferred_element_type=jnp.float32)


def _stage_a_body(x_ref, wm_ref, ws_ref, xs_ref, p_ref):
    xb = x_ref[...]
    p_ref[...] = _dot(xb, wm_ref[...])
    xs_ref[...] = _dot(xb, ws_ref[...])


def _stage_a(x, Wm1, Ws1):
    blk = _TBLK
    return pl.pallas_call(
        _stage_a_body,
        grid=(_TGRID,),
        in_specs=[
            pl.BlockSpec((blk, 84), lambda i: (i, 0)),
            pl.BlockSpec((84, _H), lambda i: (0, 0)),
            pl.BlockSpec((84, _H), lambda i: (0, 0)),
        ],
        out_specs=[pl.BlockSpec((blk, _H), lambda i: (i, 0)),
                   pl.BlockSpec((blk, _H), lambda i: (i, 0))],
        out_shape=[jax.ShapeDtypeStruct((_N, _H), jnp.float32),
                   jax.ShapeDtypeStruct((_N, _H), jnp.float32)],
    )(x, Wm1, Ws1)


def _ga_body(g_ref, wg_ref, bg_ref, out_ref):
    k = pl.program_id(0)
    part = _dot(g_ref[...], wg_ref[...])

    @pl.when(k == 0)
    def _():
        out_ref[...] = part + bg_ref[...]

    @pl.when(k != 0)
    def _():
        out_ref[...] += part


def _ga_proj(graph_attr, Wg, bg):
    kc = 1152
    return pl.pallas_call(
        _ga_body,
        grid=(10368 // kc,),
        in_specs=[
            pl.BlockSpec((_B, kc), lambda k: (0, k)),
            pl.BlockSpec((kc, _H), lambda k: (k, 0)),
            pl.BlockSpec((1, _H), lambda k: (0, 0)),
        ],
        out_specs=pl.BlockSpec((_B, _H), lambda k: (0, 0)),
        out_shape=jax.ShapeDtypeStruct((_B, _H), jnp.float32),
    )(graph_attr, Wg, bg.reshape(1, _H))


def _mid_body(agg_ref, ce0, ce1, w7_ref, wm_ref, prev_ref, res_ref,
              h_ref, p_ref, *, residual):
    ce = ce0[...] + ce1[...]
    cst = _dot(ce, w7_ref[...])
    h = jax.nn.relu(agg_ref[...] + cst + prev_ref[...])
    if residual:
        h = h + res_ref[...]
    h_ref[...] = h
    p_ref[...] = _dot(h, wm_ref[...])


def _stage_mid(agg, ce2, W7, Wm_next, prev, res, residual):
    blk = _TBLK
    body = functools.partial(_mid_body, residual=residual)
    return pl.pallas_call(
        body,
        grid=(_TGRID,),
        in_specs=[pl.BlockSpec((blk, _H), lambda i: (i, 0))] +
                 [pl.BlockSpec((blk, 16), lambda i: (i, 0))] * 2 +
                 [pl.BlockSpec((16, _H), lambda i: (0, 0)),
                  pl.BlockSpec((_H, _H), lambda i: (0, 0)),
                  pl.BlockSpec((blk, _H), lambda i: (i, 0)),
                  pl.BlockSpec((blk, _H), lambda i: (i, 0))],
        out_specs=[pl.BlockSpec((blk, _H), lambda i: (i, 0)),
                   pl.BlockSpec((blk, _H), lambda i: (i, 0))],
        out_shape=[jax.ShapeDtypeStruct((_N, _H), jnp.float32),
                   jax.ShapeDtypeStruct((_N, _H), jnp.float32)],
    )(agg, ce2[0], ce2[1], W7, Wm_next, prev, res)


def _pool_body(agg_ref, ce0, ce1, w7_ref, x2_ref, b_ref, psum_ref, cnt_ref):
    i = pl.program_id(0)
    ce = ce0[...] + ce1[...]
    cst = _dot(ce, w7_ref[...])
    x2 = x2_ref[...]
    x3 = jax.nn.relu(agg_ref[...] + cst + x2) + x2
    b = b_ref[0, 0, :]
    iot = lax.broadcasted_iota(jnp.int32, (_TBLK, _B), 1)
    oh = (b[:, None] == iot).astype(jnp.float32)
    psum = jax.lax.dot_general(oh, x3, (((0,), (0,)), ((), ())),
                               preferred_element_type=jnp.float32)
    cnt = jax.lax.dot_general(oh, jnp.ones((_TBLK, _H), jnp.float32),
                              (((0,), (0,)), ((), ())),
                              preferred_element_type=jnp.float32)

    @pl.when(i == 0)
    def _():
        psum_ref[...] = psum
        cnt_ref[...] = cnt

    @pl.when(i != 0)
    def _():
        psum_ref[...] += psum
        cnt_ref[...] += cnt


def _stage_pool(agg, ce2, W7, x2, batch3):
    blk = _TBLK
    return pl.pallas_call(
        _pool_body,
        grid=(_TGRID,),
        in_specs=[pl.BlockSpec((blk, _H), lambda i: (i, 0))] +
                 [pl.BlockSpec((blk, 16), lambda i: (i, 0))] * 2 +
                 [pl.BlockSpec((16, _H), lambda i: (0, 0)),
                  pl.BlockSpec((blk, _H), lambda i: (i, 0)),
                  pl.BlockSpec((1, 1, blk), lambda i: (i, 0, 0))],
        out_specs=[pl.BlockSpec((_B, _H), lambda i: (0, 0)),
                   pl.BlockSpec((_B, _H), lambda i: (0, 0))],
        out_shape=[jax.ShapeDtypeStruct((_B, _H), jnp.float32),
                   jax.ShapeDtypeStruct((_B, _H), jnp.float32)],
    )(agg, ce2[0], ce2[1], W7, x2, batch3)


def _head_body(psum_ref, cnt_ref, ga_ref, wcp_ref, wcg_ref, bc_ref,
               wl_ref, bl_ref, out_ref):
    cnt = jnp.maximum(cnt_ref[...], 1.0)
    pooled = psum_ref[...] / cnt
    h = jax.nn.relu(_dot(pooled, wcp_ref[...]) + _dot(ga_ref[...], wcg_ref[...])
                    + bc_ref[...])
    out_ref[...] = _dot(h, wl_ref[...]) + bl_ref[...]


def _head(psum, cnt, ga, Wc, bc, Wl, bl):
    return pl.pallas_call(
        _head_body,
        out_shape=jax.ShapeDtypeStruct((_B, 1), jnp.float32),
    )(psum, cnt, ga, Wc[:_H], Wc[_H:], bc.reshape(1, _H), Wl,
      bl.reshape(1, 1))


def kernel(x, edge_index, edge_attr, batch, graph_attr,
           Wm1, bm1, We1, be1, Ws1,
           Wm2, bm2, We2, be2,
           Wm3, bm3, We3, be3,
           Wg, bg, Wc, bc, Wl, bl):
    src = edge_index[0]
    dst = edge_index[1]

    eaT = jnp.concatenate(
        [edge_attr.T, jnp.ones((1, _E), jnp.float32),
         jnp.zeros((1, _E), jnp.float32)], axis=0)

    def w7(We, bm, be):
        return jnp.concatenate(
            [We, (bm + be).reshape(1, _H), jnp.zeros((16 - 7, _H), jnp.float32)],
            axis=0)

    W7_1, W7_2, W7_3 = w7(We1, bm1, be1), w7(We2, bm2, be2), w7(We3, bm3, be3)

    g4 = src[None, :] * 4 + jnp.arange(4, dtype=jnp.int32)[:, None]

    ce_part = _ce_scatter(eaT, dst)
    ce2 = (ce_part[:_N], ce_part[_NP:_NP + _N])

    xs, p1 = _stage_a(x, Wm1, Ws1)
    ga = _ga_proj(graph_attr, Wg, bg)

    agg1 = _scatter128(p1, g4, dst)
    x1, p2 = _stage_mid(agg1, ce2, W7_1, Wm2, xs, xs, residual=False)

    agg2 = _scatter128(p2, g4, dst)
    x2, p3 = _stage_mid(agg2, ce2, W7_2, Wm3, x1, x1, residual=True)

    agg3 = _scatter128(p3, g4, dst)
    batch3 = batch.reshape(_TGRID, 1, _TBLK)
    psum, cnt = _stage_pool(agg3, ce2, W7_3, x2, batch3)

    return _head(psum, cnt, ga, Wc, bc, Wl, bl)


# speedup vs baseline: 1.5509x; 1.0491x over previous
import functools
import jax
import jax.numpy as jnp
from jax import lax
from jax.experimental import pallas as pl
from jax.experimental.pallas import tpu as pltpu
from jax.experimental.pallas import tpu_sc as plsc

_N = 50000
_E = 800000
_B = 64
_H = 128
_CW = 32
_BK = 400
_EPT = _E // 16
_NB = _EPT // _BK
_NP = 50048
_ZR = 782
_RPT = _NP // 16
_BK2 = 1000
_EPT2 = _E // 32
_NB2 = _E // (32 * _BK2)
_TBLK = 1000
_TGRID = _N // _TBLK

_SC_PARAMS = pltpu.CompilerParams(use_tc_tiling_on_sc=False,
                                  needs_layout_passes=False)


def _sc_mesh():
    return plsc.VectorSubcoreMesh(core_axis_name="c", subcore_axis_name="s")


def _zero_fill(ref, nrow, ncol):
    z = jnp.zeros((16,), jnp.float32)

    def body(r, carry):
        for j in range(ncol // 16):
            ref[r, pl.ds(16 * j, 16)] = z
        return carry

    lax.fori_loop(0, nrow, body, 0)


def _ce_body(eaT_hbm, dst_hbm, out_hbm, dst_v, strip_v, rows_v, acc, gsem, lsem):
    cid = lax.axis_index("c")
    sid = lax.axis_index("s")
    _zero_fill(rows_v, _BK2, 16)
    base = sid * _RPT
    for z in range(3):
        pltpu.sync_copy(rows_v, acc.at[pl.ds(base + z * _BK2, _BK2)])
    pltpu.sync_copy(rows_v.at[pl.ds(0, _RPT - 3 * _BK2)],
                    acc.at[pl.ds(base + 3 * _BK2, _RPT - 3 * _BK2)])
    plsc.subcore_barrier()
    wid = cid * 16 + sid
    lane = lax.iota(jnp.int32, 16)

    def batch(i, carry):
        off = (i * 32 + wid) * _BK2
        for d in range(7):
            pltpu.async_copy(eaT_hbm.at[d, pl.ds(off, _BK2)], strip_v.at[d], lsem)
        pltpu.async_copy(dst_hbm.at[pl.ds(off, _BK2)], dst_v, lsem)
        for d in range(7):
            pltpu.make_async_copy(eaT_hbm.at[d, pl.ds(off, _BK2)],
                                  strip_v.at[d], lsem).wait()
        pltpu.make_async_copy(dst_hbm.at[pl.ds(off, _BK2)], dst_v, lsem).wait()

        def grp(g, c2):
            rowi = 16 * g + lane
            for d in range(7):
                v = strip_v[d, pl.ds(16 * g, 16)]
                plsc.store_scatter(rows_v, [rowi, jnp.full((16,), d, jnp.int32)], v)
            return c2

        lax.fori_loop(0, _BK2 // 16, grp, 0)
        if _BK2 % 16:
            tail = _BK2 % 16
            rowi = (_BK2 - 16) + lane
            msk = lane >= (16 - tail)
            for d in range(7):
                v = strip_v[d, pl.ds(_BK2 - 16, 16)]
                plsc.store_scatter(rows_v, [rowi, jnp.full((16,), d, jnp.int32)],
                                   v, mask=msk)
        pltpu.sync_copy(rows_v, acc.at[dst_v], add=True)
        return carry

    lax.fori_loop(0, _NB2, batch, 0)
    plsc.subcore_barrier()
    obase = cid * _NP + sid * _RPT
    pltpu.async_copy(acc.at[pl.ds(sid * _RPT, _RPT)],
                     out_hbm.at[pl.ds(obase, _RPT)], gsem).wait()


def _ce_scatter(eaT, dst):
    k = pl.kernel(
        _ce_body,
        out_type=jax.ShapeDtypeStruct((2 * _NP, 16), jnp.float32),
        mesh=_sc_mesh(),
        compiler_params=_SC_PARAMS,
        scratch_types=[
            pltpu.VMEM((_BK2,), jnp.int32),
            pltpu.VMEM((8, _BK2), jnp.float32),
            pltpu.VMEM((_BK2, 16), jnp.float32),
            pltpu.VMEM_SHARED((_NP, 16), jnp.float32),
            pltpu.SemaphoreType.DMA,
            pltpu.SemaphoreType.DMA,
        ],
    )
    return k(eaT, dst)


def _scat_body(pr_hbm, g4_hbm, dst_hbm, out_hbm,
               gidx_v, dst_v, rows_v, acc, gsem, isem):
    cid = lax.axis_index("c")
    sid = lax.axis_index("s")
    ebase = sid * _EPT

    def zfill0():
        z = jnp.zeros((16,), jnp.float32)

        def zbody(r, carry):
            rows_v[0, r, pl.ds(0, 16)] = z
            rows_v[0, r, pl.ds(16, 16)] = z
            return carry

        lax.fori_loop(0, _BK, zbody, 0)

    for cl in range(2):
        c = cid * 2 + cl
        zfill0()
        base = sid * _RPT
        zsrc = rows_v.at[0]
        for z in range(7):
            pltpu.sync_copy(zsrc, acc.at[pl.ds(base + z * _BK, _BK)])
        pltpu.sync_copy(zsrc.at[pl.ds(0, _RPT - 7 * _BK)],
                        acc.at[pl.ds(base + 7 * _BK, _RPT - 7 * _BK)])
        plsc.subcore_barrier()

        pltpu.sync_copy(g4_hbm.at[c, pl.ds(ebase, _BK)], gidx_v.at[0])
        pltpu.sync_copy(dst_hbm.at[pl.ds(ebase, _BK)], dst_v.at[0])
        pltpu.async_copy(pr_hbm.at[gidx_v.at[0]], rows_v.at[0], gsem)

        def batch(i, carry):
            b = lax.rem(i, 2)
            nb_ = 1 - b
            off = ebase + (i + 1) * _BK

            @pl.when(i + 1 < _NB)
            def _():
                pltpu.async_copy(g4_hbm.at[c, pl.ds(off, _BK)],
                                 gidx_v.at[nb_], isem)
                pltpu.async_copy(dst_hbm.at[pl.ds(off, _BK)], dst_v.at[nb_],
                                 isem)

            pltpu.make_async_copy(pr_hbm.at[gidx_v.at[b]], rows_v.at[b],
                                  gsem).wait()

            @pl.when(i + 1 < _NB)
            def _():
                pltpu.make_async_copy(g4_hbm.at[c, pl.ds(off, _BK)],
                                      gidx_v.at[nb_], isem).wait()
                pltpu.make_async_copy(dst_hbm.at[pl.ds(off, _BK)],
                                      dst_v.at[nb_], isem).wait()
                pltpu.async_copy(pr_hbm.at[gidx_v.at[nb_]], rows_v.at[nb_], gsem)

            pltpu.sync_copy(rows_v.at[b], acc.at[dst_v.at[b]], add=True)
            return carry

        lax.fori_loop(0, _NB, batch, 0)
        plsc.subcore_barrier()
        pltpu.sync_copy(acc.at[pl.ds(sid * _RPT, _RPT)],
                        out_hbm.at[pl.ds(sid * _RPT, _RPT), pl.ds(_CW * c, _CW)])
        plsc.subcore_barrier()


def _scatter128(p, g4, dst):
    p_r = p.reshape(4 * _N, _CW)
    k = pl.kernel(
        _scat_body,
        out_type=jax.ShapeDtypeStruct((_NP, _H), jnp.float32),
        mesh=_sc_mesh(),
        compiler_params=_SC_PARAMS,
        scratch_types=[
            pltpu.VMEM((2, _BK), jnp.int32),
            pltpu.VMEM((2, _BK), jnp.int32),
            pltpu.VMEM((2, _BK, _CW), jnp.float32),
            pltpu.VMEM_SHARED((_NP, _CW), jnp.float32),
            pltpu.SemaphoreType.DMA,
            pltpu.SemaphoreType.DMA,
        ],
    )
    return k(p_r, g4, dst)


def _dot(a, b):
    return jax.lax.dot_general(a, b, (((1,), (0,)), ((), ())),
                               preferred_element_type=jnp.float32)


def _stage_a_body(x_ref, wm_ref, ws_ref, xs_ref, p_ref):
    xb = x_ref[...]
    p_ref[...] = _dot(xb, wm_ref[...])
    xs_ref[...] = _dot(xb, ws_ref[...])


def _stage_a(x, Wm1, Ws1):
    blk = _TBLK
    return pl.pallas_call(
        _stage_a_body,
        grid=(_TGRID,),
        in_specs=[
            pl.BlockSpec((blk, 84), lambda i: (i, 0)),
            pl.BlockSpec((84, _H), lambda i: (0, 0)),
            pl.BlockSpec((84, _H), lambda i: (0, 0)),
        ],
        out_specs=[pl.BlockSpec((blk, _H), lambda i: (i, 0)),
                   pl.BlockSpec((blk, _H), lambda i: (i, 0))],
        out_shape=[jax.ShapeDtypeStruct((_N, _H), jnp.float32),
                   jax.ShapeDtypeStruct((_N, _H), jnp.float32)],
    )(x, Wm1, Ws1)


def _mid_body(agg_ref, ce0, ce1, w7_ref, wm_ref, prev_ref, res_ref,
              h_ref, p_ref, *, residual):
    ce = ce0[...] + ce1[...]
    cst = _dot(ce, w7_ref[...])
    h = jax.nn.relu(agg_ref[...] + cst + prev_ref[...])
    if residual:
        h = h + res_ref[...]
    h_ref[...] = h
    p_ref[...] = _dot(h, wm_ref[...])


def _stage_mid(agg, ce2, W7, Wm_next, prev, res, residual):
    blk = _TBLK
    body = functools.partial(_mid_body, residual=residual)
    return pl.pallas_call(
        body,
        grid=(_TGRID,),
        in_specs=[pl.BlockSpec((blk, _H), lambda i: (i, 0))] +
                 [pl.BlockSpec((blk, 16), lambda i: (i, 0))] * 2 +
                 [pl.BlockSpec((16, _H), lambda i: (0, 0)),
                  pl.BlockSpec((_H, _H), lambda i: (0, 0)),
                  pl.BlockSpec((blk, _H), lambda i: (i, 0)),
                  pl.BlockSpec((blk, _H), lambda i: (i, 0))],
        out_specs=[pl.BlockSpec((blk, _H), lambda i: (i, 0)),
                   pl.BlockSpec((blk, _H), lambda i: (i, 0))],
        out_shape=[jax.ShapeDtypeStruct((_N, _H), jnp.float32),
                   jax.ShapeDtypeStruct((_N, _H), jnp.float32)],
    )(agg, ce2[0], ce2[1], W7, Wm_next, prev, res)


def _pool_body(agg_ref, ce0, ce1, w7_ref, x2_ref, b_ref,
               g_ref, wg_ref, bg_ref, wcp_ref, wcg_ref, bc_ref, wl_ref, bl_ref,
               out_ref, psum_ref, cnt_ref):
    i = pl.program_id(0)
    ce = ce0[...] + ce1[...]
    cst = _dot(ce, w7_ref[...])
    x2 = x2_ref[...]
    x3 = jax.nn.relu(agg_ref[...] + cst + x2) + x2
    b = b_ref[0, 0, :]
    iot = lax.broadcasted_iota(jnp.int32, (_TBLK, _B), 1)
    oh = (b[:, None] == iot).astype(jnp.float32)
    psum = jax.lax.dot_general(oh, x3, (((0,), (0,)), ((), ())),
                               preferred_element_type=jnp.float32)
    cnt = jax.lax.dot_general(oh, jnp.ones((_TBLK, _H), jnp.float32),
                              (((0,), (0,)), ((), ())),
                              preferred_element_type=jnp.float32)

    @pl.when(i == 0)
    def _():
        psum_ref[...] = psum
        cnt_ref[...] = cnt

    @pl.when(i != 0)
    def _():
        psum_ref[...] += psum
        cnt_ref[...] += cnt

    @pl.when(i == _TGRID - 1)
    def _():
        ga = _dot(g_ref[...], wg_ref[...]) + bg_ref[...]
        pooled = psum_ref[...] / jnp.maximum(cnt_ref[...], 1.0)
        h = jax.nn.relu(_dot(pooled, wcp_ref[...]) + _dot(ga, wcg_ref[...])
                        + bc_ref[...])
        out_ref[...] = _dot(h, wl_ref[...]) + bl_ref[...]


def _stage_pool(agg, ce2, W7, x2, batch3, graph_attr, Wg, bg, Wc, bc, Wl, bl):
    blk = _TBLK
    out, _, _ = pl.pallas_call(
        _pool_body,
        grid=(_TGRID,),
        in_specs=[pl.BlockSpec((blk, _H), lambda i: (i, 0))] +
                 [pl.BlockSpec((blk, 16), lambda i: (i, 0))] * 2 +
                 [pl.BlockSpec((16, _H), lambda i: (0, 0)),
                  pl.BlockSpec((blk, _H), lambda i: (i, 0)),
                  pl.BlockSpec((1, 1, blk), lambda i: (i, 0, 0)),
                  pl.BlockSpec((_B, 10368), lambda i: (0, 0)),
                  pl.BlockSpec((10368, _H), lambda i: (0, 0)),
                  pl.BlockSpec((1, _H), lambda i: (0, 0)),
                  pl.BlockSpec((_H, _H), lambda i: (0, 0)),
                  pl.BlockSpec((_H, _H), lambda i: (0, 0)),
                  pl.BlockSpec((1, _H), lambda i: (0, 0)),
                  pl.BlockSpec((_H, 1), lambda i: (0, 0)),
                  pl.BlockSpec((1, 1), lambda i: (0, 0))],
        out_specs=[pl.BlockSpec((_B, 1), lambda i: (0, 0)),
                   pl.BlockSpec((_B, _H), lambda i: (0, 0)),
                   pl.BlockSpec((_B, _H), lambda i: (0, 0))],
        out_shape=[jax.ShapeDtypeStruct((_B, 1), jnp.float32),
                   jax.ShapeDtypeStruct((_B, _H), jnp.float32),
                   jax.ShapeDtypeStruct((_B, _H), jnp.float32)],
    )(agg, ce2[0], ce2[1], W7, x2, batch3, graph_attr, Wg, bg.reshape(1, _H),
      Wc[:_H], Wc[_H:], bc.reshape(1, _H), Wl, bl.reshape(1, 1))
    return out


def kernel(x, edge_index, edge_attr, batch, graph_attr,
           Wm1, bm1, We1, be1, Ws1,
           Wm2, bm2, We2, be2,
           Wm3, bm3, We3, be3,
           Wg, bg, Wc, bc, Wl, bl):
    src = edge_index[0]
    dst = edge_index[1]

    eaT = jnp.concatenate(
        [edge_attr.T, jnp.ones((1, _E), jnp.float32),
         jnp.zeros((1, _E), jnp.float32)], axis=0)

    def w7(We, bm, be):
        return jnp.concatenate(
            [We, (bm + be).reshape(1, _H), jnp.zeros((16 - 7, _H), jnp.float32)],
            axis=0)

    W7_1, W7_2, W7_3 = w7(We1, bm1, be1), w7(We2, bm2, be2), w7(We3, bm3, be3)

    g4 = src[None, :] * 4 + jnp.arange(4, dtype=jnp.int32)[:, None]

    ce_part = _ce_scatter(eaT, dst)
    ce2 = (ce_part[:_N], ce_part[_NP:_NP + _N])

    xs, p1 = _stage_a(x, Wm1, Ws1)

    agg1 = _scatter128(p1, g4, dst)
    x1, p2 = _stage_mid(agg1, ce2, W7_1, Wm2, xs, xs, residual=False)

    agg2 = _scatter128(p2, g4, dst)
    x2, p3 = _stage_mid(agg2, ce2, W7_2, Wm3, x1, x1, residual=True)

    agg3 = _scatter128(p3, g4, dst)
    batch3 = batch.reshape(_TGRID, 1, _TBLK)
    return _stage_pool(agg3, ce2, W7_3, x2, batch3, graph_attr, Wg, bg,
                       Wc, bc, Wl, bl)
